# baseline (speedup 1.0000x reference)
import functools

import jax
import jax.numpy as jnp
from jax import lax
from jax.experimental import pallas as pl
from jax.experimental.pallas import tpu as pltpu
from jax.experimental.pallas import tpu_sc as plsc

E = 320000
D = 128
N_WORKERS = 32
E_PER_W = E // N_WORKERS
CHUNK = 200
N_CHUNKS = E_PER_W // CHUNK
N_PAIRS = N_CHUNKS // 2
SC_PAD = 208
N_GROUPS = SC_PAD // 16
STAGE_W = 17


def _build_sc_kernel():
    mesh = plsc.VectorSubcoreMesh(core_axis_name="c", subcore_axis_name="s")

    @functools.partial(
        pl.kernel,
        out_type=jax.ShapeDtypeStruct((E,), jnp.float32),
        mesh=mesh,
        compiler_params=pltpu.CompilerParams(
            needs_layout_passes=False, use_tc_tiling_on_sc=False),
        scratch_types=[
            pltpu.VMEM((E_PER_W,), jnp.int32),
            pltpu.VMEM((E_PER_W,), jnp.int32),
            pltpu.VMEM((CHUNK, D // 2), jnp.int32),
            pltpu.VMEM((CHUNK, D // 2), jnp.int32),
            pltpu.VMEM((CHUNK, D // 2), jnp.int32),
            pltpu.VMEM((CHUNK, D // 2), jnp.int32),
            pltpu.VMEM((CHUNK,), jnp.float32),
            pltpu.VMEM((CHUNK,), jnp.float32),
            pltpu.VMEM((16, STAGE_W), jnp.float32),
            pltpu.VMEM((16, STAGE_W), jnp.float32),
            pltpu.SemaphoreType.DMA,
            pltpu.SemaphoreType.DMA,
            pltpu.SemaphoreType.DMA,
            pltpu.SemaphoreType.DMA,
        ],
    )
    def sc_kernel(h_hbm, src_hbm, dst_hbm, out_hbm,
                  idx_s, idx_d, rs0, rd0, rs1, rd1, sc0, sc1, stage_a, stage_b,
                  gsem0, gsem1, osem0, osem1):
        wid = lax.axis_index("s") * 2 + lax.axis_index("c")
        base0 = wid * E_PER_W
        pltpu.sync_copy(src_hbm.at[pl.ds(base0, E_PER_W)], idx_s)
        pltpu.sync_copy(dst_hbm.at[pl.ds(base0, E_PER_W)], idx_d)

        def fire_gather(ci, rs, rd, gsem):
            off = ci * CHUNK
            pltpu.async_copy(h_hbm.at[idx_s.at[pl.ds(off, CHUNK)]], rs, gsem)
            pltpu.async_copy(h_hbm.at[idx_d.at[pl.ds(off, CHUNK)]], rd, gsem)

        def wait_gather(ci, rs, rd, gsem):
            off = ci * CHUNK
            pltpu.make_async_copy(
                h_hbm.at[idx_s.at[pl.ds(off, CHUNK)]], rs, gsem).wait()
            pltpu.make_async_copy(
                h_hbm.at[idx_d.at[pl.ds(off, CHUNK)]], rd, gsem).wait()

        zeros16 = jnp.zeros((16,), jnp.float32)
        lane = lax.iota(jnp.int32, 16)

        def compute(ci, rs, rd, scb, osem, first):
            @pl.when(jnp.logical_not(first))
            def _():
                pltpu.make_async_copy(
                    scb.at[pl.ds(0, CHUNK)],
                    out_hbm.at[pl.ds(base0 + (ci - 2) * CHUNK, CHUNK)],
                    osem).wait()

            def one_group(gbase, stg):
                for el in range(16):
                    e = gbase + el
                    prods = []
                    for k in range(D // 32):
                        vs = plsc.bitcast(rs[e, pl.ds(16 * k, 16)],
                                          jnp.bfloat16)
                        vd = plsc.bitcast(rd[e, pl.ds(16 * k, 16)],
                                          jnp.bfloat16)
                        prods.append(vs * vd)
                    q0, q1 = plsc.unpack(prods[0] + prods[1],
                                         format=plsc.PackFormat.INTERLEAVED)
                    q2, q3 = plsc.unpack(prods[2] + prods[3],
                                         format=plsc.PackFormat.INTERLEAVED)
                    stg[el, pl.ds(0, 16)] = (q0 + q1) + (q2 + q3)
                sums = [zeros16, zeros16, zeros16, zeros16]
                for k in range(16):
                    col = jnp.full((16,), k, jnp.int32)
                    sums[k % 4] = sums[k % 4] + plsc.load_gather(
                        stg, [lane, col])
                scb[pl.ds(gbase, 16)] = ((sums[0] + sums[1])
                                         + (sums[2] + sums[3]))

            def group_body(g, carry):
                gbase = jnp.minimum(g * 16, CHUNK - 16)
                one_group(gbase, stage_a)
                return carry

            lax.fori_loop(0, CHUNK // 16 + 1, group_body, 0)

            pltpu.async_copy(
                scb.at[pl.ds(0, CHUNK)],
                out_hbm.at[pl.ds(base0 + ci * CHUNK, CHUNK)],
                osem)

        fire_gather(0, rs0, rd0, gsem0)

        def pair_body(g, carry):
            c0 = 2 * g
            fire_gather(c0 + 1, rs1, rd1, gsem1)
            wait_gather(c0, rs0, rd0, gsem0)
            compute(c0, rs0, rd0, sc0, osem0, g == 0)

            @pl.when(g < N_PAIRS - 1)
            def _():
                fire_gather(c0 + 2, rs0, rd0, gsem0)
            wait_gather(c0 + 1, rs1, rd1, gsem1)
            compute(c0 + 1, rs1, rd1, sc1, osem1, g == 0)
            return carry

        lax.fori_loop(0, N_PAIRS, pair_body, 0)

        pltpu.make_async_copy(
            sc0.at[pl.ds(0, CHUNK)],
            out_hbm.at[pl.ds(base0 + (N_CHUNKS - 2) * CHUNK, CHUNK)],
            osem0).wait()
        pltpu.make_async_copy(
            sc1.at[pl.ds(0, CHUNK)],
            out_hbm.at[pl.ds(base0 + (N_CHUNKS - 1) * CHUNK, CHUNK)],
            osem1).wait()

    return sc_kernel


_sc_kernel = _build_sc_kernel()


@jax.jit
def kernel(h, edge_index):
    src = edge_index[0].astype(jnp.int32)
    dst = edge_index[1].astype(jnp.int32)
    h_packed = lax.bitcast_convert_type(
        h.astype(jnp.bfloat16).reshape(h.shape[0], D // 2, 2), jnp.int32)
    return _sc_kernel(h_packed, src, dst)


# speedup vs baseline: 1.0045x; 1.0045x over previous
import functools

import jax
import jax.numpy as jnp
from jax import lax
from jax.experimental import pallas as pl
from jax.experimental.pallas import tpu as pltpu
from jax.experimental.pallas import tpu_sc as plsc

E = 320000
D = 128
N_WORKERS = 32
E_PER_W = E // N_WORKERS
CHUNK = 200
N_CHUNKS = E_PER_W // CHUNK
N_PAIRS = N_CHUNKS // 2
SC_PAD = 208
N_GROUPS = SC_PAD // 16
STAGE_W = 17


def _build_sc_kernel():
    mesh = plsc.VectorSubcoreMesh(core_axis_name="c", subcore_axis_name="s")

    @functools.partial(
        pl.kernel,
        out_type=jax.ShapeDtypeStruct((E,), jnp.float32),
        mesh=mesh,
        compiler_params=pltpu.CompilerParams(
            needs_layout_passes=False, use_tc_tiling_on_sc=False),
        scratch_types=[
            pltpu.VMEM((E_PER_W,), jnp.int32),
            pltpu.VMEM((E_PER_W,), jnp.int32),
            pltpu.VMEM((CHUNK, D // 2), jnp.int32),
            pltpu.VMEM((CHUNK, D // 2), jnp.int32),
            pltpu.VMEM((CHUNK, D // 2), jnp.int32),
            pltpu.VMEM((CHUNK, D // 2), jnp.int32),
            pltpu.VMEM((CHUNK,), jnp.float32),
            pltpu.VMEM((CHUNK,), jnp.float32),
            pltpu.VMEM((16, STAGE_W), jnp.float32),
            pltpu.VMEM((16, STAGE_W), jnp.float32),
            pltpu.SemaphoreType.DMA,
            pltpu.SemaphoreType.DMA,
            pltpu.SemaphoreType.DMA,
            pltpu.SemaphoreType.DMA,
        ],
    )
    def sc_kernel(h_hbm, src_hbm, dst_hbm, out_hbm,
                  idx_s, idx_d, rs0, rd0, rs1, rd1, sc0, sc1, stage_a, stage_b,
                  gsem0, gsem1, osem0, osem1):
        wid = lax.axis_index("s") * 2 + lax.axis_index("c")
        base0 = wid * E_PER_W
        pltpu.sync_copy(src_hbm.at[pl.ds(base0, E_PER_W)], idx_s)
        pltpu.sync_copy(dst_hbm.at[pl.ds(base0, E_PER_W)], idx_d)

        def fire_gather(ci, rs, rd, gsem):
            off = ci * CHUNK
            pltpu.async_copy(h_hbm.at[idx_s.at[pl.ds(off, CHUNK)]], rs, gsem)
            pltpu.async_copy(h_hbm.at[idx_d.at[pl.ds(off, CHUNK)]], rd, gsem)

        def wait_gather(ci, rs, rd, gsem):
            off = ci * CHUNK
            pltpu.make_async_copy(
                h_hbm.at[idx_s.at[pl.ds(off, CHUNK)]], rs, gsem).wait()
            pltpu.make_async_copy(
                h_hbm.at[idx_d.at[pl.ds(off, CHUNK)]], rd, gsem).wait()

        zeros16 = jnp.zeros((16,), jnp.float32)
        lane = lax.iota(jnp.int32, 16)

        def compute(ci, rs, rd, scb, osem, first):
            @pl.when(jnp.logical_not(first))
            def _():
                pltpu.make_async_copy(
                    scb.at[pl.ds(0, CHUNK)],
                    out_hbm.at[pl.ds(base0 + (ci - 2) * CHUNK, CHUNK)],
                    osem).wait()

            def one_group(gbase, stg):
                for el in range(16):
                    e = gbase + el
                    acc0 = acc1 = zeros16
                    for k in range(D // 32):
                        vs = plsc.bitcast(rs[e, pl.ds(16 * k, 16)],
                                          jnp.bfloat16)
                        vd = plsc.bitcast(rd[e, pl.ds(16 * k, 16)],
                                          jnp.bfloat16)
                        p0, p1 = plsc.unpack(
                            vs * vd, format=plsc.PackFormat.INTERLEAVED)
                        acc0 = acc0 + p0
                        acc1 = acc1 + p1
                    stg[el, pl.ds(0, 16)] = acc0 + acc1
                sums = [zeros16, zeros16, zeros16, zeros16]
                for k in range(16):
                    col = jnp.full((16,), k, jnp.int32)
                    sums[k % 4] = sums[k % 4] + plsc.load_gather(
                        stg, [lane, col])
                scb[pl.ds(gbase, 16)] = ((sums[0] + sums[1])
                                         + (sums[2] + sums[3]))

            def group_pair_body(p, carry):
                one_group(jnp.minimum(32 * p, CHUNK - 16), stage_a)
                one_group(jnp.minimum(32 * p + 16, CHUNK - 16), stage_b)
                return carry

            lax.fori_loop(0, (CHUNK + 31) // 32, group_pair_body, 0)

            pltpu.async_copy(
                scb.at[pl.ds(0, CHUNK)],
                out_hbm.at[pl.ds(base0 + ci * CHUNK, CHUNK)],
                osem)

        fire_gather(0, rs0, rd0, gsem0)

        def pair_body(g, carry):
            c0 = 2 * g
            fire_gather(c0 + 1, rs1, rd1, gsem1)
            wait_gather(c0, rs0, rd0, gsem0)
            compute(c0, rs0, rd0, sc0, osem0, g == 0)

            @pl.when(g < N_PAIRS - 1)
            def _():
                fire_gather(c0 + 2, rs0, rd0, gsem0)
            wait_gather(c0 + 1, rs1, rd1, gsem1)
            compute(c0 + 1, rs1, rd1, sc1, osem1, g == 0)
            return carry

        lax.fori_loop(0, N_PAIRS, pair_body, 0)

        pltpu.make_async_copy(
            sc0.at[pl.ds(0, CHUNK)],
            out_hbm.at[pl.ds(base0 + (N_CHUNKS - 2) * CHUNK, CHUNK)],
            osem0).wait()
        pltpu.make_async_copy(
            sc1.at[pl.ds(0, CHUNK)],
            out_hbm.at[pl.ds(base0 + (N_CHUNKS - 1) * CHUNK, CHUNK)],
            osem1).wait()

    return sc_kernel


_sc_kernel = _build_sc_kernel()


@jax.jit
def kernel(h, edge_index):
    src = edge_index[0].astype(jnp.int32)
    dst = edge_index[1].astype(jnp.int32)
    h_packed = lax.bitcast_convert_type(
        h.astype(jnp.bfloat16).reshape(h.shape[0], D // 2, 2), jnp.int32)
    return _sc_kernel(h_packed, src, dst)


# speedup vs baseline: 1.0579x; 1.0532x over previous
import functools

import jax
import jax.numpy as jnp
from jax import lax
from jax.experimental import pallas as pl
from jax.experimental.pallas import tpu as pltpu
from jax.experimental.pallas import tpu_sc as plsc

E = 320000
D = 128
N_WORKERS = 32
E_PER_W = E // N_WORKERS
CHUNK = 200
N_CHUNKS = E_PER_W // CHUNK
N_PAIRS = N_CHUNKS // 2
SC_PAD = 208
N_GROUPS = SC_PAD // 16
STAGE_W = 17


def _build_sc_kernel():
    mesh = plsc.VectorSubcoreMesh(core_axis_name="c", subcore_axis_name="s")

    @functools.partial(
        pl.kernel,
        out_type=jax.ShapeDtypeStruct((E,), jnp.float32),
        mesh=mesh,
        compiler_params=pltpu.CompilerParams(
            needs_layout_passes=False, use_tc_tiling_on_sc=False),
        scratch_types=[
            pltpu.VMEM((E_PER_W,), jnp.int32),
            pltpu.VMEM((E_PER_W,), jnp.int32),
            pltpu.VMEM((CHUNK, D // 2), jnp.int32),
            pltpu.VMEM((CHUNK, D // 2), jnp.int32),
            pltpu.VMEM((CHUNK, D // 2), jnp.int32),
            pltpu.VMEM((CHUNK, D // 2), jnp.int32),
            pltpu.VMEM((CHUNK,), jnp.float32),
            pltpu.VMEM((CHUNK,), jnp.float32),
            pltpu.VMEM((16, STAGE_W), jnp.float32),
            pltpu.SemaphoreType.DMA,
            pltpu.SemaphoreType.DMA,
            pltpu.SemaphoreType.DMA,
            pltpu.SemaphoreType.DMA,
        ],
    )
    def sc_kernel(h_hbm, src_hbm, dst_hbm, out_hbm,
                  idx_s, idx_d, rs0, rd0, rs1, rd1, sc0, sc1, stage,
                  gsem0, gsem1, osem0, osem1):
        wid = lax.axis_index("s") * 2 + lax.axis_index("c")
        base0 = wid * E_PER_W
        pltpu.sync_copy(src_hbm.at[pl.ds(base0, E_PER_W)], idx_s)
        pltpu.sync_copy(dst_hbm.at[pl.ds(base0, E_PER_W)], idx_d)

        def fire_gather(ci, rs, rd, gsem):
            off = ci * CHUNK
            pltpu.async_copy(h_hbm.at[idx_s.at[pl.ds(off, CHUNK)]], rs, gsem)
            pltpu.async_copy(h_hbm.at[idx_d.at[pl.ds(off, CHUNK)]], rd, gsem)

        def wait_gather(ci, rs, rd, gsem):
            off = ci * CHUNK
            pltpu.make_async_copy(
                h_hbm.at[idx_s.at[pl.ds(off, CHUNK)]], rs, gsem).wait()
            pltpu.make_async_copy(
                h_hbm.at[idx_d.at[pl.ds(off, CHUNK)]], rd, gsem).wait()

        zeros16 = jnp.zeros((16,), jnp.float32)
        lane = lax.iota(jnp.int32, 16)

        def compute(ci, rs, rd, scb, osem, first):
            @pl.when(jnp.logical_not(first))
            def _():
                pltpu.make_async_copy(
                    scb.at[pl.ds(0, CHUNK)],
                    out_hbm.at[pl.ds(base0 + (ci - 2) * CHUNK, CHUNK)],
                    osem).wait()

            def one_group(gbase, stg):
                for el in range(16):
                    e = gbase + el
                    acc0 = acc1 = zeros16
                    for k in range(D // 32):
                        vs = plsc.bitcast(rs[e, pl.ds(16 * k, 16)],
                                          jnp.bfloat16)
                        vd = plsc.bitcast(rd[e, pl.ds(16 * k, 16)],
                                          jnp.bfloat16)
                        p0, p1 = plsc.unpack(
                            vs * vd, format=plsc.PackFormat.INTERLEAVED)
                        acc0 = acc0 + p0
                        acc1 = acc1 + p1
                    stg[el, pl.ds(0, 16)] = acc0 + acc1
                sums = [zeros16, zeros16, zeros16, zeros16]
                for k in range(16):
                    col = jnp.full((16,), k, jnp.int32)
                    sums[k % 4] = sums[k % 4] + plsc.load_gather(
                        stg, [lane, col])
                scb[pl.ds(gbase, 16)] = ((sums[0] + sums[1])
                                         + (sums[2] + sums[3]))

            def group_body(g, carry):
                gbase = jnp.minimum(g * 16, CHUNK - 16)
                one_group(gbase, stage)
                return carry

            lax.fori_loop(0, CHUNK // 16 + 1, group_body, 0)

            pltpu.async_copy(
                scb.at[pl.ds(0, CHUNK)],
                out_hbm.at[pl.ds(base0 + ci * CHUNK, CHUNK)],
                osem)

        fire_gather(0, rs0, rd0, gsem0)

        def pair_body(g, carry):
            c0 = 2 * g
            fire_gather(c0 + 1, rs1, rd1, gsem1)
            wait_gather(c0, rs0, rd0, gsem0)
            compute(c0, rs0, rd0, sc0, osem0, g == 0)

            @pl.when(g < N_PAIRS - 1)
            def _():
                fire_gather(c0 + 2, rs0, rd0, gsem0)
            wait_gather(c0 + 1, rs1, rd1, gsem1)
            compute(c0 + 1, rs1, rd1, sc1, osem1, g == 0)
            return carry

        lax.fori_loop(0, N_PAIRS, pair_body, 0)

        pltpu.make_async_copy(
            sc0.at[pl.ds(0, CHUNK)],
            out_hbm.at[pl.ds(base0 + (N_CHUNKS - 2) * CHUNK, CHUNK)],
            osem0).wait()
        pltpu.make_async_copy(
            sc1.at[pl.ds(0, CHUNK)],
            out_hbm.at[pl.ds(base0 + (N_CHUNKS - 1) * CHUNK, CHUNK)],
            osem1).wait()

    return sc_kernel


_sc_kernel = _build_sc_kernel()


@jax.jit
def kernel(h, edge_index):
    src = edge_index[0].astype(jnp.int32)
    dst = edge_index[1].astype(jnp.int32)
    h_packed = lax.bitcast_convert_type(
        h.astype(jnp.bfloat16).reshape(h.shape[0], D // 2, 2), jnp.int32)
    return _sc_kernel(h_packed, src, dst)
